# baseline (speedup 1.0000x reference)
import functools

import jax
import jax.numpy as jnp
from jax.experimental import pallas as pl
from jax.experimental.pallas import tpu as pltpu

_K = 1024
_D = 64
_BETA = 0.25
_BS = 512


def _vq_body(z_ref, e_ref, zq_ref, idx_ref, counts_ref, lossrow_ref):
    b = pl.program_id(0)
    j = pl.program_id(1)

    zb = z_ref[0]
    e = e_ref[...]

    esq = jnp.sum(e * e, axis=1, keepdims=True)
    zsq = jnp.sum(zb * zb, axis=0, keepdims=True)
    mm2 = jax.lax.dot_general(
        e * (-2.0), zb, (((1,), (0,)), ((), ())),
        preferred_element_type=jnp.float32)
    dist = (zsq + esq) + mm2

    kiota = jax.lax.broadcasted_iota(jnp.int32, dist.shape, 0).astype(jnp.float32)
    minv = jnp.min(dist, axis=0, keepdims=True)
    cand = jnp.where(dist == minv, kiota, float(_K))
    idx_f = jnp.min(cand, axis=0, keepdims=True)
    idx_ref[0] = idx_f.astype(jnp.int32)

    oh = jnp.where(kiota == idx_f, 1.0, 0.0)
    zq = jax.lax.dot_general(
        e, oh, (((0,), (0,)), ((), ())),
        preferred_element_type=jnp.float32)
    zq_ref[0] = zb + (zq - zb)

    ones = jnp.ones((_BS, 1), jnp.float32)
    part_counts = jax.lax.dot_general(
        oh, ones, (((1,), (0,)), ((), ())),
        preferred_element_type=jnp.float32)

    first = jnp.logical_and(b == 0, j == 0)

    @pl.when(first)
    def _init():
        counts_ref[...] = part_counts
        lossrow_ref[...] = minv

    @pl.when(jnp.logical_not(first))
    def _acc():
        counts_ref[...] = counts_ref[...] + part_counts
        lossrow_ref[...] = lossrow_ref[...] + minv


def _fin_body(counts_ref, lossrow_ref, loss_ref, perp_ref):
    n_total = 65536
    avg = counts_ref[...] * (1.0 / n_total)
    ent = jnp.sum(avg * jnp.log(avg + 1e-10)).reshape(1, 1)
    perp_ref[...] = jnp.exp(-ent)
    loss_ref[...] = jnp.sum(lossrow_ref[...]).reshape(1, 1) * (
        _BETA / (n_total * _D))


@functools.partial(jax.jit, static_argnames=("interpret",))
def _vq(z, embedding, interpret=False):
    bsz, c, dd, h, w = z.shape
    s = dd * h * w
    zr = z.reshape(bsz, c, s)
    nj = s // _BS
    grid = (bsz, nj)

    zq, idxb, counts, lossrow = pl.pallas_call(
        _vq_body,
        interpret=interpret,
        grid=grid,
        in_specs=[
            pl.BlockSpec((1, c, _BS), lambda b, j: (b, 0, j)),
            pl.BlockSpec((_K, _D), lambda b, j: (0, 0)),
        ],
        out_specs=[
            pl.BlockSpec((1, c, _BS), lambda b, j: (b, 0, j)),
            pl.BlockSpec((1, 1, _BS), lambda b, j: (b * nj + j, 0, 0)),
            pl.BlockSpec((_K, 1), lambda b, j: (0, 0)),
            pl.BlockSpec((1, _BS), lambda b, j: (0, 0)),
        ],
        out_shape=[
            jax.ShapeDtypeStruct((bsz, c, s), jnp.float32),
            jax.ShapeDtypeStruct((bsz * nj, 1, _BS), jnp.int32),
            jax.ShapeDtypeStruct((_K, 1), jnp.float32),
            jax.ShapeDtypeStruct((1, _BS), jnp.float32),
        ],
    )(zr, embedding)

    loss, perp = pl.pallas_call(
        _fin_body,
        interpret=interpret,
        out_shape=[
            jax.ShapeDtypeStruct((1, 1), jnp.float32),
            jax.ShapeDtypeStruct((1, 1), jnp.float32),
        ],
    )(counts.reshape(8, 128), lossrow)

    z_q_out = zq.reshape(bsz, c, dd, h, w)
    indices = idxb.reshape(bsz, dd, h, w)
    return z_q_out, loss[0, 0], indices, perp[0, 0]


def kernel(z, embedding):
    return _vq(z, embedding)


# speedup vs baseline: 1.1689x; 1.1689x over previous
import functools

import jax
import jax.numpy as jnp
from jax.experimental import pallas as pl
from jax.experimental.pallas import tpu as pltpu

_K = 1024
_D = 64
_BETA = 0.25
_BS = 1024


def _vq_body(z_ref, e_ref, zq_ref, idx_ref, counts_ref, lossrow_ref):
    b = pl.program_id(0)
    j = pl.program_id(1)

    zb = z_ref[0]
    e = e_ref[...]

    esq = jnp.sum(e * e, axis=1, keepdims=True)
    zsq = jnp.sum(zb * zb, axis=0, keepdims=True)
    mm2 = jax.lax.dot_general(
        e * (-2.0), zb, (((1,), (0,)), ((), ())),
        preferred_element_type=jnp.float32)
    dist = (zsq + esq) + mm2

    kiota = jax.lax.broadcasted_iota(jnp.int32, dist.shape, 0).astype(jnp.float32)
    minv = jnp.min(dist, axis=0, keepdims=True)
    cand = jnp.where(dist == minv, kiota, float(_K))
    idx_f = jnp.min(cand, axis=0, keepdims=True)
    idx_ref[0] = idx_f.astype(jnp.int32)

    oh = jnp.where(kiota == idx_f, 1.0, 0.0)
    zq = jax.lax.dot_general(
        e, oh, (((0,), (0,)), ((), ())),
        preferred_element_type=jnp.float32)
    zq_ref[0] = zb + (zq - zb)

    ones = jnp.ones((_BS, 1), jnp.float32)
    part_counts = jax.lax.dot_general(
        oh, ones, (((1,), (0,)), ((), ())),
        preferred_element_type=jnp.float32)

    first = jnp.logical_and(b == 0, j == 0)

    @pl.when(first)
    def _init():
        counts_ref[...] = part_counts
        lossrow_ref[...] = minv

    @pl.when(jnp.logical_not(first))
    def _acc():
        counts_ref[...] = counts_ref[...] + part_counts
        lossrow_ref[...] = lossrow_ref[...] + minv


def _fin_body(counts_ref, lossrow_ref, loss_ref, perp_ref):
    n_total = 65536
    avg = counts_ref[...] * (1.0 / n_total)
    ent = jnp.sum(avg * jnp.log(avg + 1e-10)).reshape(1, 1)
    perp_ref[...] = jnp.exp(-ent)
    loss_ref[...] = jnp.sum(lossrow_ref[...]).reshape(1, 1) * (
        _BETA / (n_total * _D))


@functools.partial(jax.jit, static_argnames=("interpret",))
def _vq(z, embedding, interpret=False):
    bsz, c, dd, h, w = z.shape
    s = dd * h * w
    zr = z.reshape(bsz, c, s)
    nj = s // _BS
    grid = (bsz, nj)

    zq, idxb, counts, lossrow = pl.pallas_call(
        _vq_body,
        interpret=interpret,
        grid=grid,
        in_specs=[
            pl.BlockSpec((1, c, _BS), lambda b, j: (b, 0, j)),
            pl.BlockSpec((_K, _D), lambda b, j: (0, 0)),
        ],
        out_specs=[
            pl.BlockSpec((1, c, _BS), lambda b, j: (b, 0, j)),
            pl.BlockSpec((1, 1, _BS), lambda b, j: (b * nj + j, 0, 0)),
            pl.BlockSpec((_K, 1), lambda b, j: (0, 0)),
            pl.BlockSpec((1, _BS), lambda b, j: (0, 0)),
        ],
        out_shape=[
            jax.ShapeDtypeStruct((bsz, c, s), jnp.float32),
            jax.ShapeDtypeStruct((bsz * nj, 1, _BS), jnp.int32),
            jax.ShapeDtypeStruct((_K, 1), jnp.float32),
            jax.ShapeDtypeStruct((1, _BS), jnp.float32),
        ],
    )(zr, embedding)

    loss, perp = pl.pallas_call(
        _fin_body,
        interpret=interpret,
        out_shape=[
            jax.ShapeDtypeStruct((1, 1), jnp.float32),
            jax.ShapeDtypeStruct((1, 1), jnp.float32),
        ],
    )(counts.reshape(8, 128), lossrow)

    z_q_out = zq.reshape(bsz, c, dd, h, w)
    indices = idxb.reshape(bsz, dd, h, w)
    return z_q_out, loss[0, 0], indices, perp[0, 0]


def kernel(z, embedding):
    return _vq(z, embedding)


# speedup vs baseline: 1.1878x; 1.0162x over previous
import functools

import jax
import jax.numpy as jnp
from jax.experimental import pallas as pl
from jax.experimental.pallas import tpu as pltpu

_K = 1024
_D = 64
_BETA = 0.25
_BS = 2048


def _vq_body(z_ref, e_ref, zq_ref, idx_ref, counts_ref, lossrow_ref):
    b = pl.program_id(0)
    j = pl.program_id(1)

    zb = z_ref[0]
    e = e_ref[...]

    esq = jnp.sum(e * e, axis=1, keepdims=True)
    zsq = jnp.sum(zb * zb, axis=0, keepdims=True)
    mm2 = jax.lax.dot_general(
        e * (-2.0), zb, (((1,), (0,)), ((), ())),
        preferred_element_type=jnp.float32)
    dist = (zsq + esq) + mm2

    kiota = jax.lax.broadcasted_iota(jnp.int32, dist.shape, 0).astype(jnp.float32)
    minv = jnp.min(dist, axis=0, keepdims=True)
    cand = jnp.where(dist == minv, kiota, float(_K))
    idx_f = jnp.min(cand, axis=0, keepdims=True)
    idx_ref[0] = idx_f.astype(jnp.int32)

    oh = jnp.where(kiota == idx_f, 1.0, 0.0)
    zq = jax.lax.dot_general(
        e, oh, (((0,), (0,)), ((), ())),
        preferred_element_type=jnp.float32)
    zq_ref[0] = zb + (zq - zb)

    ones = jnp.ones((_BS, 1), jnp.float32)
    part_counts = jax.lax.dot_general(
        oh, ones, (((1,), (0,)), ((), ())),
        preferred_element_type=jnp.float32)

    first = jnp.logical_and(b == 0, j == 0)

    @pl.when(first)
    def _init():
        counts_ref[...] = part_counts
        lossrow_ref[...] = minv

    @pl.when(jnp.logical_not(first))
    def _acc():
        counts_ref[...] = counts_ref[...] + part_counts
        lossrow_ref[...] = lossrow_ref[...] + minv


def _fin_body(counts_ref, lossrow_ref, loss_ref, perp_ref):
    n_total = 65536
    avg = counts_ref[...] * (1.0 / n_total)
    ent = jnp.sum(avg * jnp.log(avg + 1e-10)).reshape(1, 1)
    perp_ref[...] = jnp.exp(-ent)
    loss_ref[...] = jnp.sum(lossrow_ref[...]).reshape(1, 1) * (
        _BETA / (n_total * _D))


@functools.partial(jax.jit, static_argnames=("interpret",))
def _vq(z, embedding, interpret=False):
    bsz, c, dd, h, w = z.shape
    s = dd * h * w
    zr = z.reshape(bsz, c, s)
    nj = s // _BS
    grid = (bsz, nj)

    zq, idxb, counts, lossrow = pl.pallas_call(
        _vq_body,
        interpret=interpret,
        grid=grid,
        in_specs=[
            pl.BlockSpec((1, c, _BS), lambda b, j: (b, 0, j)),
            pl.BlockSpec((_K, _D), lambda b, j: (0, 0)),
        ],
        out_specs=[
            pl.BlockSpec((1, c, _BS), lambda b, j: (b, 0, j)),
            pl.BlockSpec((1, 1, _BS), lambda b, j: (b * nj + j, 0, 0)),
            pl.BlockSpec((_K, 1), lambda b, j: (0, 0)),
            pl.BlockSpec((1, _BS), lambda b, j: (0, 0)),
        ],
        out_shape=[
            jax.ShapeDtypeStruct((bsz, c, s), jnp.float32),
            jax.ShapeDtypeStruct((bsz * nj, 1, _BS), jnp.int32),
            jax.ShapeDtypeStruct((_K, 1), jnp.float32),
            jax.ShapeDtypeStruct((1, _BS), jnp.float32),
        ],
    )(zr, embedding)

    loss, perp = pl.pallas_call(
        _fin_body,
        interpret=interpret,
        out_shape=[
            jax.ShapeDtypeStruct((1, 1), jnp.float32),
            jax.ShapeDtypeStruct((1, 1), jnp.float32),
        ],
    )(counts.reshape(8, 128), lossrow)

    z_q_out = zq.reshape(bsz, c, dd, h, w)
    indices = idxb.reshape(bsz, dd, h, w)
    return z_q_out, loss[0, 0], indices, perp[0, 0]


def kernel(z, embedding):
    return _vq(z, embedding)


# speedup vs baseline: 1.2585x; 1.0595x over previous
import functools

import jax
import jax.numpy as jnp
from jax.experimental import pallas as pl
from jax.experimental.pallas import tpu as pltpu

_K = 1024
_D = 64
_BETA = 0.25
_BS = 2048


def _vq_body(z_ref, e_ref, zq_ref, idx_ref, counts_ref, lossrow_ref):
    b = pl.program_id(0)
    j = pl.program_id(1)

    zb = z_ref[0]
    e = e_ref[...]

    esq = jnp.sum(e * e, axis=1, keepdims=True)
    zsq = jnp.sum(zb * zb, axis=0, keepdims=True)
    mm2 = jax.lax.dot_general(
        e * (-2.0), zb, (((1,), (0,)), ((), ())),
        preferred_element_type=jnp.float32)
    dist = (zsq + esq) + mm2

    kiota = jax.lax.broadcasted_iota(jnp.int32, dist.shape, 0).astype(jnp.float32)
    minv = jnp.min(dist, axis=0, keepdims=True)
    cand = jnp.where(dist == minv, kiota, float(_K))
    idx_f = jnp.min(cand, axis=0, keepdims=True)
    idx_ref[0] = idx_f.astype(jnp.int32)

    oh = jnp.where(kiota == idx_f, 1.0, 0.0)
    zq = jax.lax.dot_general(
        e, oh, (((0,), (0,)), ((), ())),
        preferred_element_type=jnp.float32)
    zq_ref[0] = zb + (zq - zb)

    ones = jnp.ones((1, _BS), jnp.float32)
    part_counts = jax.lax.dot_general(
        ones, oh, (((1,), (1,)), ((), ())),
        preferred_element_type=jnp.float32)

    first = jnp.logical_and(b == 0, j == 0)

    @pl.when(first)
    def _init():
        counts_ref[...] = part_counts
        lossrow_ref[...] = minv

    @pl.when(jnp.logical_not(first))
    def _acc():
        counts_ref[...] = counts_ref[...] + part_counts
        lossrow_ref[...] = lossrow_ref[...] + minv


def _fin_body(counts_ref, lossrow_ref, loss_ref, perp_ref):
    n_total = 65536
    avg = counts_ref[...] * (1.0 / n_total)
    ent = jnp.sum(avg * jnp.log(avg + 1e-10)).reshape(1, 1)
    perp_ref[...] = jnp.exp(-ent)
    loss_ref[...] = jnp.sum(lossrow_ref[...]).reshape(1, 1) * (
        _BETA / (n_total * _D))


@functools.partial(jax.jit, static_argnames=("interpret",))
def _vq(z, embedding, interpret=False):
    bsz, c, dd, h, w = z.shape
    s = dd * h * w
    zr = z.reshape(bsz, c, s)
    nj = s // _BS
    grid = (bsz, nj)

    zq, idxb, counts, lossrow = pl.pallas_call(
        _vq_body,
        interpret=interpret,
        grid=grid,
        in_specs=[
            pl.BlockSpec((1, c, _BS), lambda b, j: (b, 0, j)),
            pl.BlockSpec((_K, _D), lambda b, j: (0, 0)),
        ],
        out_specs=[
            pl.BlockSpec((1, c, _BS), lambda b, j: (b, 0, j)),
            pl.BlockSpec((1, 1, _BS), lambda b, j: (b * nj + j, 0, 0)),
            pl.BlockSpec((1, _K), lambda b, j: (0, 0)),
            pl.BlockSpec((1, _BS), lambda b, j: (0, 0)),
        ],
        out_shape=[
            jax.ShapeDtypeStruct((bsz, c, s), jnp.float32),
            jax.ShapeDtypeStruct((bsz * nj, 1, _BS), jnp.int32),
            jax.ShapeDtypeStruct((1, _K), jnp.float32),
            jax.ShapeDtypeStruct((1, _BS), jnp.float32),
        ],
    )(zr, embedding)

    loss, perp = pl.pallas_call(
        _fin_body,
        interpret=interpret,
        out_shape=[
            jax.ShapeDtypeStruct((1, 1), jnp.float32),
            jax.ShapeDtypeStruct((1, 1), jnp.float32),
        ],
    )(counts.reshape(8, 128), lossrow)

    z_q_out = zq.reshape(bsz, c, dd, h, w)
    indices = idxb.reshape(bsz, dd, h, w)
    return z_q_out, loss[0, 0], indices, perp[0, 0]


def kernel(z, embedding):
    return _vq(z, embedding)


# speedup vs baseline: 1.2678x; 1.0073x over previous
import functools

import jax
import jax.numpy as jnp
from jax.experimental import pallas as pl
from jax.experimental.pallas import tpu as pltpu

_K = 1024
_D = 64
_BETA = 0.25
_BS = 4096


def _vq_body(z_ref, e_ref, zq_ref, idx_ref, counts_ref, lossrow_ref):
    b = pl.program_id(0)
    j = pl.program_id(1)

    zb = z_ref[0]
    e = e_ref[...]

    esq = jnp.sum(e * e, axis=1, keepdims=True)
    zsq = jnp.sum(zb * zb, axis=0, keepdims=True)
    mm2 = jax.lax.dot_general(
        e * (-2.0), zb, (((1,), (0,)), ((), ())),
        preferred_element_type=jnp.float32)
    dist = (zsq + esq) + mm2

    kiota = jax.lax.broadcasted_iota(jnp.int32, dist.shape, 0).astype(jnp.float32)
    minv = jnp.min(dist, axis=0, keepdims=True)
    cand = jnp.where(dist == minv, kiota, float(_K))
    idx_f = jnp.min(cand, axis=0, keepdims=True)
    idx_ref[0] = idx_f.astype(jnp.int32)

    oh = jnp.where(kiota == idx_f, 1.0, 0.0)
    zq = jax.lax.dot_general(
        e, oh, (((0,), (0,)), ((), ())),
        preferred_element_type=jnp.float32)
    zq_ref[0] = zb + (zq - zb)

    ones = jnp.ones((1, _BS), jnp.float32)
    part_counts = jax.lax.dot_general(
        ones, oh, (((1,), (1,)), ((), ())),
        preferred_element_type=jnp.float32)

    first = jnp.logical_and(b == 0, j == 0)

    @pl.when(first)
    def _init():
        counts_ref[...] = part_counts
        lossrow_ref[...] = minv

    @pl.when(jnp.logical_not(first))
    def _acc():
        counts_ref[...] = counts_ref[...] + part_counts
        lossrow_ref[...] = lossrow_ref[...] + minv


def _fin_body(counts_ref, lossrow_ref, loss_ref, perp_ref):
    n_total = 65536
    avg = counts_ref[...] * (1.0 / n_total)
    ent = jnp.sum(avg * jnp.log(avg + 1e-10)).reshape(1, 1)
    perp_ref[...] = jnp.exp(-ent)
    loss_ref[...] = jnp.sum(lossrow_ref[...]).reshape(1, 1) * (
        _BETA / (n_total * _D))


@functools.partial(jax.jit, static_argnames=("interpret",))
def _vq(z, embedding, interpret=False):
    bsz, c, dd, h, w = z.shape
    s = dd * h * w
    zr = z.reshape(bsz, c, s)
    nj = s // _BS
    grid = (bsz, nj)

    zq, idxb, counts, lossrow = pl.pallas_call(
        _vq_body,
        interpret=interpret,
        grid=grid,
        in_specs=[
            pl.BlockSpec((1, c, _BS), lambda b, j: (b, 0, j)),
            pl.BlockSpec((_K, _D), lambda b, j: (0, 0)),
        ],
        out_specs=[
            pl.BlockSpec((1, c, _BS), lambda b, j: (b, 0, j)),
            pl.BlockSpec((1, 1, _BS), lambda b, j: (b * nj + j, 0, 0)),
            pl.BlockSpec((1, _K), lambda b, j: (0, 0)),
            pl.BlockSpec((1, _BS), lambda b, j: (0, 0)),
        ],
        out_shape=[
            jax.ShapeDtypeStruct((bsz, c, s), jnp.float32),
            jax.ShapeDtypeStruct((bsz * nj, 1, _BS), jnp.int32),
            jax.ShapeDtypeStruct((1, _K), jnp.float32),
            jax.ShapeDtypeStruct((1, _BS), jnp.float32),
        ],
    )(zr, embedding)

    loss, perp = pl.pallas_call(
        _fin_body,
        interpret=interpret,
        out_shape=[
            jax.ShapeDtypeStruct((1, 1), jnp.float32),
            jax.ShapeDtypeStruct((1, 1), jnp.float32),
        ],
    )(counts.reshape(8, 128), lossrow)

    z_q_out = zq.reshape(bsz, c, dd, h, w)
    indices = idxb.reshape(bsz, dd, h, w)
    return z_q_out, loss[0, 0], indices, perp[0, 0]


def kernel(z, embedding):
    return _vq(z, embedding)
